# baseline (speedup 1.0000x reference)
import jax
import jax.numpy as jnp
from jax.experimental import pallas as pl
from jax.experimental.pallas import tpu as pltpu

_DIM = 64
_KS = 1024
_DEPTH = 4
_NGRP = _KS // 128


def _vq_kernel(x_ref, cb_ref, cbt_ref, out_ref):
    r = x_ref[0]
    cb = cb_ref[...]
    nt = r.shape[1]
    s = jnp.zeros_like(r)
    for _ in range(_DEPTH):
        ip = jax.lax.dot_general(
            cb, r, (((1,), (0,)), ((), ())),
            preferred_element_type=jnp.float32)
        code = jnp.argmax(ip, axis=0)
        u = jnp.max(ip, axis=0)
        lane = jnp.broadcast_to((code & 127)[None, :], (_DIM, nt))
        grp = code >> 7
        g = jnp.zeros((_DIM, nt), jnp.float32)
        for q in range(_NGRP):
            cand = jnp.take_along_axis(
                cbt_ref[:, q * 128:(q + 1) * 128], lane, axis=1)
            g = jnp.where((grp == q)[None, :], cand, g)
        comp = g * u[None, :]
        s = s + comp
        r = r - comp
    out_ref[0] = s


def kernel(x, code_book):
    B, C, H, W = x.shape
    xf = x.reshape(B, C, H * W)
    out = pl.pallas_call(
        _vq_kernel,
        grid=(B,),
        in_specs=[
            pl.BlockSpec((1, C, H * W), lambda b: (b, 0, 0)),
            pl.BlockSpec((_KS, _DIM), lambda b: (0, 0)),
            pl.BlockSpec((_DIM, _KS), lambda b: (0, 0)),
        ],
        out_specs=pl.BlockSpec((1, C, H * W), lambda b: (b, 0, 0)),
        out_shape=jax.ShapeDtypeStruct((B, C, H * W), x.dtype),
        compiler_params=pltpu.CompilerParams(
            dimension_semantics=("parallel",)),
    )(xf, code_book, code_book.T)
    return out.reshape(B, C, H, W)


# speedup vs baseline: 1.7590x; 1.7590x over previous
import jax
import jax.numpy as jnp
from jax.experimental import pallas as pl
from jax.experimental.pallas import tpu as pltpu

_DIM = 64
_KS = 1024
_DEPTH = 4
_NGRP = _KS // 128
_CHUNK = 8
_NSLAB = 4
_BB = 4
_FUSE = 2


def _scan_slab(ip_slab, slice_base, nt):
    nsl = ip_slab.shape[0] // 8
    v = ip_slab.reshape(nsl, 8, nt)
    partials = []
    for c in range(0, nsl, _CHUNK):
        mc = v[c]
        kc = None
        for i in range(1, _CHUNK):
            gt = v[c + i] > mc
            mc = jnp.maximum(mc, v[c + i])
            kc = jnp.where(gt, slice_base + c + i,
                           slice_base + c if kc is None else kc)
        partials.append((mc, kc))
    return partials


def _finish(partials, cbt_ref, nt):
    vs = [p[0] for p in partials]
    ks = [p[1] for p in partials]
    while len(vs) > 1:
        nvs, nks = [], []
        for a in range(0, len(vs), 2):
            gt = vs[a + 1] > vs[a]
            nvs.append(jnp.maximum(vs[a], vs[a + 1]))
            nks.append(jnp.where(gt, ks[a + 1], ks[a]))
        vs, ks = nvs, nks
    m, kwin = vs[0], ks[0]
    row = kwin * 8 + jax.lax.broadcasted_iota(jnp.int32, (8, nt), 0)
    for sh in (4, 2, 1):
        mo = pltpu.roll(m, sh, 0)
        ro = pltpu.roll(row, sh, 0)
        better = (mo > m) | ((mo == m) & (ro < row))
        m = jnp.where(better, mo, m)
        row = jnp.where(better, ro, row)
    u64 = jnp.tile(m, (_DIM // 8, 1))
    lane = jnp.tile(row & 127, (_DIM // 8, 1))
    grp = row >> 7
    cands = [
        jnp.take_along_axis(cbt_ref[:, q * 128:(q + 1) * 128], lane, axis=1)
        for q in range(_NGRP)
    ]
    bit = 1
    while len(cands) > 1:
        mask = jnp.tile((grp & bit) != 0, (_DIM // 8, 1))
        cands = [jnp.where(mask, cands[a + 1], cands[a])
                 for a in range(0, len(cands), 2)]
        bit <<= 1
    return cands[0] * u64


def _vq_kernel(x_ref, cb_ref, cbt_ref, out_ref):
    cb = cb_ref[...]
    bb = x_ref.shape[0]
    hw = x_ref.shape[2]
    npair = bb // _FUSE
    nt = hw * _FUSE
    rows = _KS // _NSLAB
    rs = [
        jnp.concatenate(
            [x_ref[p * _FUSE + j] for j in range(_FUSE)], axis=1)
        for p in range(npair)
    ]
    ss = [jnp.zeros_like(rs[p]) for p in range(npair)]
    for _ in range(_DEPTH):
        slabs = [
            [
                jax.lax.dot_general(
                    cb[sl * rows:(sl + 1) * rows], rs[p],
                    (((1,), (0,)), ((), ())),
                    preferred_element_type=jnp.float32)
                for sl in range(_NSLAB)
            ]
            for p in range(npair)
        ]
        for p in range(npair):
            partials = []
            for sl in range(_NSLAB):
                partials += _scan_slab(slabs[p][sl], sl * rows // 8, nt)
            comp = _finish(partials, cbt_ref, nt)
            ss[p] = ss[p] + comp
            rs[p] = rs[p] - comp
    for p in range(npair):
        for j in range(_FUSE):
            out_ref[p * _FUSE + j] = ss[p][:, j * hw:(j + 1) * hw]


def kernel(x, code_book):
    B, C, H, W = x.shape
    xf = x.reshape(B, C, H * W)
    out = pl.pallas_call(
        _vq_kernel,
        grid=(B // _BB,),
        in_specs=[
            pl.BlockSpec((_BB, C, H * W), lambda b: (b, 0, 0)),
            pl.BlockSpec((_KS, _DIM), lambda b: (0, 0)),
            pl.BlockSpec((_DIM, _KS), lambda b: (0, 0)),
        ],
        out_specs=pl.BlockSpec((_BB, C, H * W), lambda b: (b, 0, 0)),
        out_shape=jax.ShapeDtypeStruct((B, C, H * W), x.dtype),
        compiler_params=pltpu.CompilerParams(
            dimension_semantics=("parallel",)),
    )(xf, code_book, code_book.T)
    return out.reshape(B, C, H, W)


# speedup vs baseline: 1.7856x; 1.0151x over previous
import jax
import jax.numpy as jnp
from jax.experimental import pallas as pl
from jax.experimental.pallas import tpu as pltpu

_DIM = 64
_KS = 1024
_DEPTH = 4
_NGRP = _KS // 128
_CHUNK = 8
_NSLAB = 2
_BB = 4
_FUSE = 2


def _scan_slab(ip_slab, slice_base, nt):
    nsl = ip_slab.shape[0] // 8
    v = ip_slab.reshape(nsl, 8, nt)
    partials = []
    for c in range(0, nsl, _CHUNK):
        mc = v[c]
        kc = None
        for i in range(1, _CHUNK):
            gt = v[c + i] > mc
            mc = jnp.maximum(mc, v[c + i])
            kc = jnp.where(gt, slice_base + c + i,
                           slice_base + c if kc is None else kc)
        partials.append((mc, kc))
    return partials


def _finish(partials, cbt_ref, nt):
    vs = [p[0] for p in partials]
    ks = [p[1] for p in partials]
    while len(vs) > 1:
        nvs, nks = [], []
        for a in range(0, len(vs), 2):
            gt = vs[a + 1] > vs[a]
            nvs.append(jnp.maximum(vs[a], vs[a + 1]))
            nks.append(jnp.where(gt, ks[a + 1], ks[a]))
        vs, ks = nvs, nks
    m, kwin = vs[0], ks[0]
    row = kwin * 8 + jax.lax.broadcasted_iota(jnp.int32, (8, nt), 0)
    for sh in (4, 2, 1):
        mo = pltpu.roll(m, sh, 0)
        ro = pltpu.roll(row, sh, 0)
        better = (mo > m) | ((mo == m) & (ro < row))
        m = jnp.where(better, mo, m)
        row = jnp.where(better, ro, row)
    u64 = jnp.tile(m, (_DIM // 8, 1))
    lane = jnp.tile(row & 127, (_DIM // 8, 1))
    grp = row >> 7
    cands = [
        jnp.take_along_axis(cbt_ref[:, q * 128:(q + 1) * 128], lane, axis=1)
        for q in range(_NGRP)
    ]
    bit = 1
    while len(cands) > 1:
        mask = jnp.tile((grp & bit) != 0, (_DIM // 8, 1))
        cands = [jnp.where(mask, cands[a + 1], cands[a])
                 for a in range(0, len(cands), 2)]
        bit <<= 1
    return cands[0] * u64


def _vq_kernel(x_ref, cb_ref, cbt_ref, out_ref):
    cb = cb_ref[...]
    bb = x_ref.shape[0]
    hw = x_ref.shape[2]
    npair = bb // _FUSE
    nt = hw * _FUSE
    rows = _KS // _NSLAB
    rs = [
        jnp.concatenate(
            [x_ref[p * _FUSE + j] for j in range(_FUSE)], axis=1)
        for p in range(npair)
    ]
    ss = [jnp.zeros_like(rs[p]) for p in range(npair)]
    for _ in range(_DEPTH):
        slabs = [
            [
                jax.lax.dot_general(
                    cb[sl * rows:(sl + 1) * rows], rs[p],
                    (((1,), (0,)), ((), ())),
                    preferred_element_type=jnp.float32)
                for sl in range(_NSLAB)
            ]
            for p in range(npair)
        ]
        partials = [[] for _ in range(npair)]
        for sl in range(_NSLAB):
            for p in range(npair):
                partials[p] += _scan_slab(slabs[p][sl], sl * rows // 8, nt)
        for p in range(npair):
            comp = _finish(partials[p], cbt_ref, nt)
            ss[p] = ss[p] + comp
            rs[p] = rs[p] - comp
    for p in range(npair):
        for j in range(_FUSE):
            out_ref[p * _FUSE + j] = ss[p][:, j * hw:(j + 1) * hw]


def kernel(x, code_book):
    B, C, H, W = x.shape
    xf = x.reshape(B, C, H * W)
    out = pl.pallas_call(
        _vq_kernel,
        grid=(B // _BB,),
        in_specs=[
            pl.BlockSpec((_BB, C, H * W), lambda b: (b, 0, 0)),
            pl.BlockSpec((_KS, _DIM), lambda b: (0, 0)),
            pl.BlockSpec((_DIM, _KS), lambda b: (0, 0)),
        ],
        out_specs=pl.BlockSpec((_BB, C, H * W), lambda b: (b, 0, 0)),
        out_shape=jax.ShapeDtypeStruct((B, C, H * W), x.dtype),
        compiler_params=pltpu.CompilerParams(
            dimension_semantics=("parallel",)),
    )(xf, code_book, code_book.T)
    return out.reshape(B, C, H, W)
